# baseline (speedup 1.0000x reference)
import jax
import jax.numpy as jnp
from jax.experimental import pallas as pl

B, D, H, L, P, C, CD, A = 4096, 128, 512, 256, 16, 8, 8, 1000
AP = 1024
BB = 1024
NEG = -1e9


def _first_argmax(x, width):
    m = jnp.max(x, axis=1, keepdims=True)
    iota = jax.lax.broadcasted_iota(jnp.int32, x.shape, 1)
    return jnp.min(jnp.where(x == m, iota, width), axis=1, keepdims=True)


def _block_kernel(obs_ref, encW_ref, encb_ref, bbW_ref, bbb_ref, muW_ref,
                  mub_ref, coW_ref, cob_ref, salW_ref, salb_ref, actW_ref,
                  gdW_ref, gdb_ref, rfW_ref, rgW_ref, dhW_ref, auxW_ref,
                  auxb_ref, full_ref, sa_ref, pred_ref):
    f32 = jnp.float32
    dot = lambda a, b: jnp.dot(a, b, preferred_element_type=f32)

    obs = obs_ref[...]
    h1 = jnp.maximum(dot(obs, encW_ref[...]) + encb_ref[...], 0.0)
    h = jnp.tanh(dot(h1, bbW_ref[...]) + bbb_ref[...])
    z = dot(h, muW_ref[...]) + mub_ref[...]
    coords = dot(z, coW_ref[...]) + cob_ref[...]
    sal = dot(z, salW_ref[...]) + salb_ref[...]
    geom = dot(z, gdW_ref[...]) + gdb_ref[...]
    delta = dot(h, rfW_ref[...])
    gate = jax.nn.sigmoid(dot(h, rgW_ref[...])[:, :1])
    refined = geom + gate * delta
    q = dot(h, dhW_ref[...])
    pred_ref[...] = dot(z, auxW_ref[...]) + auxb_ref[...]

    a_iota = jax.lax.broadcasted_iota(jnp.int32, (BB, AP), 1)
    valid_a = a_iota < A
    actW = actW_ref[...]
    p_iota = jax.lax.broadcasted_iota(jnp.int32, (BB, P), 1)
    pool_of_col = jax.lax.broadcasted_iota(jnp.int32, (BB, P * CD), 1) // CD

    out = jnp.full((BB, AP), NEG, dtype=f32)
    msal = sal
    for c in range(C):
        cur_max = jnp.max(msal, axis=1, keepdims=True)
        cur_idx = _first_argmax(msal, P)
        cc = jnp.zeros((BB, CD), dtype=f32)
        for p in range(P):
            cc = cc + jnp.where(cur_idx == p,
                                coords[:, p * CD:(p + 1) * CD], 0.0)
        scores = jnp.where(valid_a, dot(cc, actW), -jnp.inf)
        ca = _first_argmax(scores, AP)
        rpc = refined[:, c * CD:(c + 1) * CD] + cc
        t = [q[:, dd:dd + 1] * rpc[:, dd:dd + 1] for dd in range(CD)]
        ll = (((t[0] + t[4]) + (t[2] + t[6]))
              + ((t[1] + t[5]) + (t[3] + t[7])))
        fill = jnp.where(cur_max > 0.0, ll, NEG)
        out = jnp.where(a_iota == ca, fill, out)
        msal = jnp.where(p_iota == cur_idx, -jnp.inf, msal)

    full_ref[...] = out
    sa_ref[...] = _first_argmax(jnp.where(valid_a, out, -jnp.inf), AP)


def kernel(obs, enc_W, enc_b, bb_W, bb_b, mu_W, mu_b, co_W, co_b, sal_W,
           sal_b, act_W, gd_W, gd_b, rf_W, rg_W, dh_W, aux_W, aux_b):
    actWp = jnp.pad(act_W, ((0, 0), (0, AP - A)))
    args = (
        obs,
        enc_W, enc_b.reshape(1, H),
        bb_W, bb_b.reshape(1, H),
        mu_W, mu_b.reshape(1, L),
        co_W, co_b.reshape(1, P * CD),
        sal_W, sal_b.reshape(1, P),
        actWp,
        gd_W, gd_b.reshape(1, C * CD),
        rf_W, jnp.pad(rg_W, ((0, 0), (0, 127))),
        dh_W, aux_W, aux_b.reshape(1, D),
    )

    def _bcast(shape):
        return pl.BlockSpec(shape, lambda i: (0,) * len(shape))

    in_specs = [pl.BlockSpec((BB, D), lambda i: (i, 0))]
    in_specs += [_bcast(a.shape) for a in args[1:]]

    out_shapes = (
        jax.ShapeDtypeStruct((B, AP), jnp.float32),
        jax.ShapeDtypeStruct((B, 1), jnp.int32),
        jax.ShapeDtypeStruct((B, D), jnp.float32),
    )
    out_specs = (
        pl.BlockSpec((BB, AP), lambda i: (i, 0)),
        pl.BlockSpec((BB, 1), lambda i: (i, 0)),
        pl.BlockSpec((BB, D), lambda i: (i, 0)),
    )

    full_p, sa, pred = pl.pallas_call(
        _block_kernel,
        grid=(B // BB,),
        in_specs=in_specs,
        out_specs=out_specs,
        out_shape=out_shapes,
    )(*args)

    return full_p[:, :A], sa.reshape(B), pred


# speedup vs baseline: 1.2757x; 1.2757x over previous
import jax
import jax.numpy as jnp
from jax.experimental import pallas as pl

B, D, H, L, P, C, CD, A = 4096, 128, 512, 256, 16, 8, 8, 1000
AP = 1024
BB = 512
NEG = -1e9


def _first_argmax(x, width):
    m = jnp.max(x, axis=1, keepdims=True)
    iota = jax.lax.broadcasted_iota(jnp.int32, x.shape, 1)
    return jnp.min(jnp.where(x == m, iota, width), axis=1, keepdims=True)


def _block_kernel(obs_ref, encW_ref, encb_ref, bbW_ref, bbb_ref, muW_ref,
                  mub_ref, coW_ref, cob_ref, salW_ref, salb_ref, actW_ref,
                  gdW_ref, gdb_ref, rfW_ref, rgW_ref, dhW_ref, auxW_ref,
                  auxb_ref, full_ref, sa_ref, pred_ref):
    f32 = jnp.float32
    dot = lambda a, b: jnp.dot(a, b, preferred_element_type=f32)

    obs = obs_ref[...]
    h1 = jnp.maximum(dot(obs, encW_ref[...]) + encb_ref[...], 0.0)
    h = jnp.tanh(dot(h1, bbW_ref[...]) + bbb_ref[...])
    z = dot(h, muW_ref[...]) + mub_ref[...]
    coords = dot(z, coW_ref[...]) + cob_ref[...]
    sal = dot(z, salW_ref[...]) + salb_ref[...]
    geom = dot(z, gdW_ref[...]) + gdb_ref[...]
    delta = dot(h, rfW_ref[...])
    gate = jax.nn.sigmoid(dot(h, rgW_ref[...])[:, :1])
    refined = geom + gate * delta
    q = dot(h, dhW_ref[...])
    pred_ref[...] = dot(z, auxW_ref[...]) + auxb_ref[...]

    a_iota = jax.lax.broadcasted_iota(jnp.int32, (BB, AP), 1)
    valid_a = a_iota < A
    actW = actW_ref[...]
    p_iota = jax.lax.broadcasted_iota(jnp.int32, (BB, P), 1)
    pool_of_col = jax.lax.broadcasted_iota(jnp.int32, (BB, P * CD), 1) // CD

    out = jnp.full((BB, AP), NEG, dtype=f32)
    msal = sal
    for c in range(C):
        cur_max = jnp.max(msal, axis=1, keepdims=True)
        cur_idx = _first_argmax(msal, P)
        csel = jnp.where(pool_of_col == cur_idx, coords, 0.0)
        r = csel + jnp.roll(csel, -64, axis=1)
        r = r + jnp.roll(r, -32, axis=1)
        r = r + jnp.roll(r, -16, axis=1)
        r = r + jnp.roll(r, -CD, axis=1)
        cc = r[:, :CD]
        scores = jnp.where(valid_a, dot(cc, actW), -jnp.inf)
        ca = _first_argmax(scores, AP)
        rpc = refined[:, c * CD:(c + 1) * CD] + cc
        t = [q[:, dd:dd + 1] * rpc[:, dd:dd + 1] for dd in range(CD)]
        ll = (((t[0] + t[4]) + (t[2] + t[6]))
              + ((t[1] + t[5]) + (t[3] + t[7])))
        fill = jnp.where(cur_max > 0.0, ll, NEG)
        out = jnp.where(a_iota == ca, fill, out)
        msal = jnp.where(p_iota == cur_idx, -jnp.inf, msal)

    full_ref[...] = out
    sa_ref[...] = _first_argmax(jnp.where(valid_a, out, -jnp.inf), AP)


def kernel(obs, enc_W, enc_b, bb_W, bb_b, mu_W, mu_b, co_W, co_b, sal_W,
           sal_b, act_W, gd_W, gd_b, rf_W, rg_W, dh_W, aux_W, aux_b):
    actWp = jnp.pad(act_W, ((0, 0), (0, AP - A)))
    args = (
        obs,
        enc_W, enc_b.reshape(1, H),
        bb_W, bb_b.reshape(1, H),
        mu_W, mu_b.reshape(1, L),
        co_W, co_b.reshape(1, P * CD),
        sal_W, sal_b.reshape(1, P),
        actWp,
        gd_W, gd_b.reshape(1, C * CD),
        rf_W, jnp.pad(rg_W, ((0, 0), (0, 127))),
        dh_W, aux_W, aux_b.reshape(1, D),
    )

    def _bcast(shape):
        return pl.BlockSpec(shape, lambda i: (0,) * len(shape))

    in_specs = [pl.BlockSpec((BB, D), lambda i: (i, 0))]
    in_specs += [_bcast(a.shape) for a in args[1:]]

    out_shapes = (
        jax.ShapeDtypeStruct((B, AP), jnp.float32),
        jax.ShapeDtypeStruct((B, 1), jnp.int32),
        jax.ShapeDtypeStruct((B, D), jnp.float32),
    )
    out_specs = (
        pl.BlockSpec((BB, AP), lambda i: (i, 0)),
        pl.BlockSpec((BB, 1), lambda i: (i, 0)),
        pl.BlockSpec((BB, D), lambda i: (i, 0)),
    )

    full_p, sa, pred = pl.pallas_call(
        _block_kernel,
        grid=(B // BB,),
        in_specs=in_specs,
        out_specs=out_specs,
        out_shape=out_shapes,
    )(*args)

    return full_p[:, :A], sa.reshape(B), pred


# speedup vs baseline: 1.5756x; 1.2351x over previous
import jax
import jax.numpy as jnp
from jax.experimental import pallas as pl

B, D, H, L, P, C, CD, A = 4096, 128, 512, 256, 16, 8, 8, 1000
AP = 1024
BB = 1024
NEG = -1e9


def _first_argmax(x, width):
    m = jnp.max(x, axis=1, keepdims=True)
    iota = jax.lax.broadcasted_iota(jnp.int32, x.shape, 1)
    return jnp.min(jnp.where(x == m, iota, width), axis=1, keepdims=True)


def _block_kernel(obs_ref, encW_ref, encb_ref, bbW_ref, bbb_ref, muW_ref,
                  mub_ref, coW_ref, cob_ref, salW_ref, salb_ref, actW_ref,
                  gdW_ref, gdb_ref, rfW_ref, rgW_ref, dhW_ref, auxW_ref,
                  auxb_ref, full_ref, sa_ref, pred_ref):
    f32 = jnp.float32
    dot = lambda a, b: jnp.dot(a, b, preferred_element_type=f32)

    obs = obs_ref[...]
    h1 = jnp.maximum(dot(obs, encW_ref[...]) + encb_ref[...], 0.0)
    h = jnp.tanh(dot(h1, bbW_ref[...]) + bbb_ref[...])
    z = dot(h, muW_ref[...]) + mub_ref[...]
    coords = dot(z, coW_ref[...]) + cob_ref[...]
    sal = dot(z, salW_ref[...]) + salb_ref[...]
    geom = dot(z, gdW_ref[...]) + gdb_ref[...]
    delta = dot(h, rfW_ref[...])
    gate = jax.nn.sigmoid(dot(h, rgW_ref[...])[:, :1])
    refined = geom + gate * delta
    q = dot(h, dhW_ref[...])
    pred_ref[...] = dot(z, auxW_ref[...]) + auxb_ref[...]

    a_iota = jax.lax.broadcasted_iota(jnp.int32, (BB, AP), 1)
    valid_a = a_iota < A
    actW = actW_ref[...]
    p_iota = jax.lax.broadcasted_iota(jnp.int32, (BB, P), 1)
    pool_of_col = jax.lax.broadcasted_iota(jnp.int32, (BB, P * CD), 1) // CD

    out = jnp.full((BB, AP), NEG, dtype=f32)
    msal = sal
    for c in range(C):
        cur_max = jnp.max(msal, axis=1, keepdims=True)
        cur_idx = _first_argmax(msal, P)
        csel = jnp.where(pool_of_col == cur_idx, coords, 0.0)
        r = csel + jnp.roll(csel, -64, axis=1)
        r = r + jnp.roll(r, -32, axis=1)
        r = r + jnp.roll(r, -16, axis=1)
        r = r + jnp.roll(r, -CD, axis=1)
        cc = r[:, :CD]
        scores = jnp.where(valid_a, dot(cc, actW), -jnp.inf)
        ca = _first_argmax(scores, AP)
        rpc = refined[:, c * CD:(c + 1) * CD] + cc
        t = [q[:, dd:dd + 1] * rpc[:, dd:dd + 1] for dd in range(CD)]
        ll = (((t[0] + t[4]) + (t[2] + t[6]))
              + ((t[1] + t[5]) + (t[3] + t[7])))
        fill = jnp.where(cur_max > 0.0, ll, NEG)
        out = jnp.where(a_iota == ca, fill, out)
        msal = jnp.where(p_iota == cur_idx, -jnp.inf, msal)

    full_ref[...] = out
    sa_ref[...] = _first_argmax(jnp.where(valid_a, out, -jnp.inf), AP)


def kernel(obs, enc_W, enc_b, bb_W, bb_b, mu_W, mu_b, co_W, co_b, sal_W,
           sal_b, act_W, gd_W, gd_b, rf_W, rg_W, dh_W, aux_W, aux_b):
    actWp = jnp.pad(act_W, ((0, 0), (0, AP - A)))
    args = (
        obs,
        enc_W, enc_b.reshape(1, H),
        bb_W, bb_b.reshape(1, H),
        mu_W, mu_b.reshape(1, L),
        co_W, co_b.reshape(1, P * CD),
        sal_W, sal_b.reshape(1, P),
        actWp,
        gd_W, gd_b.reshape(1, C * CD),
        rf_W, jnp.pad(rg_W, ((0, 0), (0, 127))),
        dh_W, aux_W, aux_b.reshape(1, D),
    )

    def _bcast(shape):
        return pl.BlockSpec(shape, lambda i: (0,) * len(shape))

    in_specs = [pl.BlockSpec((BB, D), lambda i: (i, 0))]
    in_specs += [_bcast(a.shape) for a in args[1:]]

    out_shapes = (
        jax.ShapeDtypeStruct((B, AP), jnp.float32),
        jax.ShapeDtypeStruct((B, 1), jnp.int32),
        jax.ShapeDtypeStruct((B, D), jnp.float32),
    )
    out_specs = (
        pl.BlockSpec((BB, AP), lambda i: (i, 0)),
        pl.BlockSpec((BB, 1), lambda i: (i, 0)),
        pl.BlockSpec((BB, D), lambda i: (i, 0)),
    )

    full_p, sa, pred = pl.pallas_call(
        _block_kernel,
        grid=(B // BB,),
        in_specs=in_specs,
        out_specs=out_specs,
        out_shape=out_shapes,
    )(*args)

    return full_p[:, :A], sa.reshape(B), pred
